# Initial kernel scaffold; baseline (speedup 1.0000x reference)
import jax
import jax.numpy as jnp
from jax.experimental import pallas as pl


def kernel(x, edge_index, Wenc, benc, We1, be1, We2, be2, We3, be3, Wn1, bn1, Wn2, bn2):
    raise NotImplementedError("write your pallas kernel here")


# speedup vs baseline: 1.3546x; 1.3546x over previous
import functools

import jax
import jax.numpy as jnp
from jax import lax
from jax.experimental import pallas as pl
from jax.experimental.pallas import tpu as pltpu
from jax.experimental.pallas import tpu_sc as plsc

F_IN = 128
HID = 64
D = F_IN + HID
HD = D // 2
N_ITERS = 4

NC = 2
NS = 16
NW = NC * NS
CB = 128


def _split_tables(h2):
    rb = h2.shape[0]
    zpad = jnp.zeros((rb, 128 - HD), jnp.float32)
    lo = jnp.concatenate([h2[:, :HD], zpad], axis=1)
    hi = jnp.concatenate([h2[:, HD:], zpad], axis=1)
    return jnp.stack([lo, hi], axis=0)


def _encode_body(x_ref, wenc_ref, benc_ref, w1_ref, h_ref, ab_ref, tbl_ref):
    x = x_ref[...]
    henc = jnp.maximum(
        jnp.dot(x, wenc_ref[...], preferred_element_type=jnp.float32)
        + benc_ref[...], 0.0)
    h = jnp.concatenate([henc, x], axis=1)
    h_ref[...] = h
    ab_ref[...] = jnp.dot(h, w1_ref[...], preferred_element_type=jnp.float32)
    tbl_ref[...] = _split_tables(h)


def _make_encode(n, rb):
    grid = n // rb
    full = lambda shape: pl.BlockSpec(shape, lambda i: (0,) * len(shape))
    return pl.pallas_call(
        _encode_body,
        grid=(grid,),
        in_specs=[
            pl.BlockSpec((rb, F_IN), lambda i: (i, 0)),
            full((F_IN, HID)), full((1, HID)),
            full((D, 2 * HID)),
        ],
        out_specs=[
            pl.BlockSpec((rb, D), lambda i: (i, 0)),
            pl.BlockSpec((rb, 2 * HID), lambda i: (i, 0)),
            pl.BlockSpec((2, rb, 128), lambda i: (0, i, 0)),
        ],
        out_shape=[
            jax.ShapeDtypeStruct((n, D), jnp.float32),
            jax.ShapeDtypeStruct((n, 2 * HID), jnp.float32),
            jax.ShapeDtypeStruct((2, n, 128), jnp.float32),
        ],
    )


def _edge_body(sigmoid, sa_ref, sb_ref, be1_ref, w2_ref, be2_ref, w3_ref,
               be3_ref, mask_ref, out_ref):
    z1 = jnp.maximum(
        sa_ref[:, :HID] + sb_ref[:, HID:] + be1_ref[...], 0.0)
    z2 = jnp.maximum(
        jnp.dot(z1, w2_ref[...], preferred_element_type=jnp.float32)
        + be2_ref[...], 0.0)
    logit = jnp.sum(z2 * w3_ref[...], axis=1, keepdims=True) + be3_ref[...]
    if sigmoid:
        out_ref[...] = jax.nn.sigmoid(logit) * mask_ref[...]
    else:
        out_ref[...] = logit


def _make_edge(epad, rb, sigmoid):
    grid = epad // rb
    full = lambda shape: pl.BlockSpec(shape, lambda i: (0,) * len(shape))
    return pl.pallas_call(
        functools.partial(_edge_body, sigmoid),
        grid=(grid,),
        in_specs=[
            pl.BlockSpec((rb, 2 * HID), lambda i: (i, 0)),
            pl.BlockSpec((rb, 2 * HID), lambda i: (i, 0)),
            full((1, HID)), full((HID, HID)), full((1, HID)),
            full((1, HID)), full((1, 1)),
            pl.BlockSpec((rb, 1), lambda i: (i, 0)),
        ],
        out_specs=pl.BlockSpec((rb, 1), lambda i: (i, 0)),
        out_shape=jax.ShapeDtypeStruct((epad, 1), jnp.float32),
    )


def _node_body(h_ref, p_ref, x_ref, wn1s_ref, wn1a_ref, bn1_ref, wn2_ref,
               bn2_ref, w1_ref, h2_ref, ab_ref, tbl_ref):
    h = h_ref[...]
    aggr = jnp.concatenate([p_ref[0, :, :HD], p_ref[1, :, :HD]], axis=1)
    n1 = jnp.maximum(
        jnp.dot(h, wn1s_ref[...], preferred_element_type=jnp.float32)
        + jnp.dot(aggr, wn1a_ref[...], preferred_element_type=jnp.float32)
        + bn1_ref[...], 0.0)
    hn = jnp.dot(n1, wn2_ref[...], preferred_element_type=jnp.float32) + bn2_ref[...]
    h2 = jnp.concatenate([hn, x_ref[...]], axis=1) + h
    h2_ref[...] = h2
    ab_ref[...] = jnp.dot(h2, w1_ref[...], preferred_element_type=jnp.float32)
    tbl_ref[...] = _split_tables(h2)


def _make_node(n, rb):
    grid = n // rb
    full = lambda shape: pl.BlockSpec(shape, lambda i: (0,) * len(shape))
    return pl.pallas_call(
        _node_body,
        grid=(grid,),
        in_specs=[
            pl.BlockSpec((rb, D), lambda i: (i, 0)),
            pl.BlockSpec((2, rb, 128), lambda i: (0, i, 0)),
            pl.BlockSpec((rb, F_IN), lambda i: (i, 0)),
            full((D, HID)), full((D, HID)), full((1, HID)),
            full((HID, HID)), full((1, HID)),
            full((D, 2 * HID)),
        ],
        out_specs=[
            pl.BlockSpec((rb, D), lambda i: (i, 0)),
            pl.BlockSpec((rb, 2 * HID), lambda i: (i, 0)),
            pl.BlockSpec((2, rb, 128), lambda i: (0, i, 0)),
        ],
        out_shape=[
            jax.ShapeDtypeStruct((n, D), jnp.float32),
            jax.ShapeDtypeStruct((n, 2 * HID), jnp.float32),
            jax.ShapeDtypeStruct((2, n, 128), jnp.float32),
        ],
    )


def _make_sc_gather(ch):
    mesh = plsc.VectorSubcoreMesh(core_axis_name="c", subcore_axis_name="s",
                                  num_cores=NC, num_subcores=NS)

    @functools.partial(
        pl.kernel,
        out_type=(
            jax.ShapeDtypeStruct((NW, ch, CB, 2 * HID), jnp.float32),
            jax.ShapeDtypeStruct((NW, ch, CB, 2 * HID), jnp.float32),
        ),
        mesh=mesh,
        scratch_types=[
            pltpu.VMEM((ch, CB), jnp.int32),
            pltpu.VMEM((ch, CB), jnp.int32),
            pltpu.VMEM((CB, 2 * HID), jnp.float32),
            pltpu.VMEM((CB, 2 * HID), jnp.float32),
            pltpu.SemaphoreType.DMA,
            pltpu.SemaphoreType.DMA,
        ],
    )
    def gather(idxs_hbm, idxe_hbm, ab_hbm, oa_hbm, ob_hbm,
               vidx_s, vidx_e, ra, rb, sem_a, sem_b):
        cid = lax.axis_index("c")
        sid = lax.axis_index("s")
        wid = sid * NC + cid
        pltpu.sync_copy(idxs_hbm.at[wid], vidx_s)
        pltpu.sync_copy(idxe_hbm.at[wid], vidx_e)

        def chunk(j, carry):
            cpa = pltpu.async_copy(ab_hbm.at[vidx_s.at[j]], ra, sem_a)
            cpb = pltpu.async_copy(ab_hbm.at[vidx_e.at[j]], rb, sem_b)
            cpa.wait()
            cpb.wait()
            pltpu.sync_copy(ra, oa_hbm.at[wid, j])
            pltpu.sync_copy(rb, ob_hbm.at[wid, j])
            return carry

        lax.fori_loop(0, ch, chunk, 0)

    return gather


SG = 16


def _make_sc_scatter(ch2, n):
    mesh = plsc.VectorSubcoreMesh(core_axis_name="c", subcore_axis_name="s",
                                  num_cores=NC, num_subcores=NS)
    npad = -(-n // (NS * CB)) * NS * CB
    rows_per_tile = npad // NS
    zchunk = CB

    @functools.partial(
        pl.kernel,
        out_type=jax.ShapeDtypeStruct((NC, npad, 128), jnp.float32),
        mesh=mesh,
        scratch_types=[
            pltpu.VMEM((SG, CB), jnp.int32),
            pltpu.VMEM((SG, CB), jnp.int32),
            pltpu.VMEM((SG, CB), jnp.float32),
            pltpu.VMEM((CB, 128), jnp.float32),
            pltpu.VMEM_SHARED((npad, 128), jnp.float32),
            pltpu.SemaphoreType.DMA,
        ],
    )
    def scatter(idxs_hbm, idxe_hbm, w_hbm, tbl_hbm, out_hbm,
                vidx_s, vidx_e, vw, rows, accum, sem):
        cid = lax.axis_index("c")
        sid = lax.axis_index("s")
        tbl = tbl_hbm.at[cid]

        zv = jnp.zeros((16,), jnp.float32)

        def zrow(r, carry):
            for c in range(128 // 16):
                rows[r, pl.ds(c * 16, 16)] = zv
            return carry

        lax.fori_loop(0, zchunk, zrow, 0)
        base = sid * rows_per_tile
        for q in range(rows_per_tile // zchunk):
            pltpu.sync_copy(rows.at[pl.ds(0, zchunk)],
                            accum.at[pl.ds(base + q * zchunk, zchunk)])
        plsc.subcore_barrier()

        def scale(j):
            def per_group(g, carry):
                wv = vw[j, pl.ds(g * 16, 16)]
                for i in range(16):
                    e = g * 16 + i
                    w = wv[i]
                    for c in range(128 // 16):
                        sl = pl.ds(c * 16, 16)
                        rows[e, sl] = rows[e, sl] * w
                return carry
            lax.fori_loop(0, CB // 16, per_group, 0)

        def chunk(j, carry):
            pltpu.async_copy(tbl.at[vidx_e.at[j]], rows, sem).wait()
            scale(j)
            pltpu.sync_copy(rows, accum.at[vidx_s.at[j]], add=True)
            pltpu.async_copy(tbl.at[vidx_s.at[j]], rows, sem).wait()
            scale(j)
            pltpu.sync_copy(rows, accum.at[vidx_e.at[j]], add=True)
            return carry

        def slab(s, carry):
            pltpu.sync_copy(idxs_hbm.at[sid, pl.ds(s * SG, SG)], vidx_s)
            pltpu.sync_copy(idxe_hbm.at[sid, pl.ds(s * SG, SG)], vidx_e)
            pltpu.sync_copy(w_hbm.at[sid, pl.ds(s * SG, SG)], vw)
            lax.fori_loop(0, SG, chunk, 0)
            return carry

        lax.fori_loop(0, ch2 // SG, slab, 0)
        plsc.subcore_barrier()

        for q in range(rows_per_tile // zchunk):
            r0 = base + q * zchunk
            pltpu.sync_copy(accum.at[pl.ds(r0, zchunk)],
                            rows.at[pl.ds(0, zchunk)])
            pltpu.sync_copy(rows.at[pl.ds(0, zchunk)],
                            out_hbm.at[cid, pl.ds(r0, zchunk)])

    return scatter


def kernel(x, edge_index, Wenc, benc, We1, be1, We2, be2, We3, be3,
           Wn1, bn1, Wn2, bn2):
    n = x.shape[0]
    e = edge_index.shape[1]
    chunks_total = -(-e // (CB * NW * SG // 2)) * (NW * SG // 2)
    epad = chunks_total * CB
    ch = chunks_total // NW
    ch2 = chunks_total // NS

    pad = epad - e
    start = jnp.concatenate([edge_index[0], jnp.zeros((pad,), jnp.int32)])
    end = jnp.concatenate([edge_index[1], jnp.zeros((pad,), jnp.int32)])
    idx_s = start.reshape(NW, ch, CB)
    idx_e = end.reshape(NW, ch, CB)
    idx_s2 = start.reshape(NS, ch2, CB)
    idx_e2 = end.reshape(NS, ch2, CB)
    mask = (jnp.arange(epad, dtype=jnp.int32) < e).astype(jnp.float32)
    mask = mask.reshape(epad, 1)

    benc_r = benc.reshape(1, HID)
    be1_r = be1.reshape(1, HID)
    be2_r = be2.reshape(1, HID)
    be3_r = be3.reshape(1, 1)
    bn1_r = bn1.reshape(1, HID)
    bn2_r = bn2.reshape(1, HID)
    we3_r = We3.reshape(1, HID)
    W1 = jnp.concatenate([We1[:D], We1[D:]], axis=1)
    Wn1_s, Wn1_a = Wn1[:D], Wn1[D:]

    encode = _make_encode(n, 2000)
    edge_w = _make_edge(epad, 4096, sigmoid=True)
    edge_logit = _make_edge(epad, 4096, sigmoid=False)
    node = _make_node(n, 2000)
    sc_gather = _make_sc_gather(ch)
    sc_scatter = _make_sc_scatter(ch2, n)

    h, ab, tbl = encode(x, Wenc, benc_r, W1)

    for _ in range(N_ITERS):
        sa, sb = sc_gather(idx_s, idx_e, ab)
        w = edge_w(sa.reshape(epad, 2 * HID), sb.reshape(epad, 2 * HID),
                   be1_r, We2, be2_r, we3_r, be3_r, mask)
        parts = sc_scatter(idx_s2, idx_e2, w.reshape(NS, ch2, CB), tbl)
        h, ab, tbl = node(h, parts, x, Wn1_s, Wn1_a, bn1_r, Wn2, bn2_r, W1)

    sa, sb = sc_gather(idx_s, idx_e, ab)
    logits = edge_logit(sa.reshape(epad, 2 * HID), sb.reshape(epad, 2 * HID),
                        be1_r, We2, be2_r, we3_r, be3_r, mask)
    return logits.reshape(epad)[:e]


# speedup vs baseline: 2.2527x; 1.6631x over previous
import functools

import jax
import jax.numpy as jnp
from jax import lax
from jax.experimental import pallas as pl
from jax.experimental.pallas import tpu as pltpu
from jax.experimental.pallas import tpu_sc as plsc

F_IN = 128
HID = 64
D = F_IN + HID
HD = D // 2
N_ITERS = 4

NC = 2
NS = 16
NW = NC * NS
CB = 128
SG = 16
GL = 5

_SC_PARAMS = pltpu.CompilerParams(use_tc_tiling_on_sc=False,
                                  needs_layout_passes=False)


def _split_tables(h2):
    return jnp.stack([h2[:, :HD], h2[:, HD:]], axis=0)


def _encode_body(x_ref, wenc_ref, benc_ref, wa_ref, wb_ref,
                 h_ref, a_ref, b_ref, tbl_ref):
    x = x_ref[...]
    henc = jnp.maximum(
        jnp.dot(x, wenc_ref[...], preferred_element_type=jnp.float32)
        + benc_ref[...], 0.0)
    h = jnp.concatenate([henc, x], axis=1)
    h_ref[...] = h
    a_ref[...] = jnp.dot(h, wa_ref[...], preferred_element_type=jnp.float32)
    b_ref[...] = jnp.dot(h, wb_ref[...], preferred_element_type=jnp.float32)
    tbl_ref[...] = _split_tables(h)


def _make_encode(n, rb):
    grid = n // rb
    full = lambda shape: pl.BlockSpec(shape, lambda i: (0,) * len(shape))
    return pl.pallas_call(
        _encode_body,
        grid=(grid,),
        in_specs=[
            pl.BlockSpec((rb, F_IN), lambda i: (i, 0)),
            full((F_IN, HID)), full((1, HID)),
            full((D, HID)), full((D, HID)),
        ],
        out_specs=[
            pl.BlockSpec((rb, D), lambda i: (i, 0)),
            pl.BlockSpec((rb, HID), lambda i: (i, 0)),
            pl.BlockSpec((rb, HID), lambda i: (i, 0)),
            pl.BlockSpec((2, rb, HD), lambda i: (0, i, 0)),
        ],
        out_shape=[
            jax.ShapeDtypeStruct((n, D), jnp.float32),
            jax.ShapeDtypeStruct((n, HID), jnp.float32),
            jax.ShapeDtypeStruct((n, HID), jnp.float32),
            jax.ShapeDtypeStruct((2, n, HD), jnp.float32),
        ],
    )


def _edge_body(sigmoid, s_ref, be1_ref, w2_ref, be2_ref, w3_ref,
               be3_ref, mask_ref, out_ref):
    z1 = jnp.maximum(s_ref[...] + be1_ref[...], 0.0)
    z2 = jnp.maximum(
        jnp.dot(z1, w2_ref[...], preferred_element_type=jnp.float32)
        + be2_ref[...], 0.0)
    logit = jnp.sum(z2 * w3_ref[...], axis=1, keepdims=True) + be3_ref[...]
    if sigmoid:
        out_ref[...] = jax.nn.sigmoid(logit) * mask_ref[...]
    else:
        out_ref[...] = logit


def _make_edge(epad, rb, sigmoid):
    grid = epad // rb
    full = lambda shape: pl.BlockSpec(shape, lambda i: (0,) * len(shape))
    return pl.pallas_call(
        functools.partial(_edge_body, sigmoid),
        grid=(grid,),
        in_specs=[
            pl.BlockSpec((rb, HID), lambda i: (i, 0)),
            full((1, HID)), full((HID, HID)), full((1, HID)),
            full((1, HID)), full((1, 1)),
            pl.BlockSpec((rb, 1), lambda i: (i, 0)),
        ],
        out_specs=pl.BlockSpec((rb, 1), lambda i: (i, 0)),
        out_shape=jax.ShapeDtypeStruct((epad, 1), jnp.float32),
    )


def _node_body(h_ref, p_ref, x_ref, wn1s_ref, wn1a_ref, bn1_ref, wn2_ref,
               bn2_ref, wa_ref, wb_ref, h2_ref, a_ref, b_ref, tbl_ref):
    h = h_ref[...]
    aggr = jnp.concatenate([p_ref[0], p_ref[1]], axis=1)
    n1 = jnp.maximum(
        jnp.dot(h, wn1s_ref[...], preferred_element_type=jnp.float32)
        + jnp.dot(aggr, wn1a_ref[...], preferred_element_type=jnp.float32)
        + bn1_ref[...], 0.0)
    hn = jnp.dot(n1, wn2_ref[...], preferred_element_type=jnp.float32) + bn2_ref[...]
    h2 = jnp.concatenate([hn, x_ref[...]], axis=1) + h
    h2_ref[...] = h2
    a_ref[...] = jnp.dot(h2, wa_ref[...], preferred_element_type=jnp.float32)
    b_ref[...] = jnp.dot(h2, wb_ref[...], preferred_element_type=jnp.float32)
    tbl_ref[...] = _split_tables(h2)


def _make_node(n, npad, rb):
    grid = n // rb
    full = lambda shape: pl.BlockSpec(shape, lambda i: (0,) * len(shape))
    return pl.pallas_call(
        _node_body,
        grid=(grid,),
        in_specs=[
            pl.BlockSpec((rb, D), lambda i: (i, 0)),
            pl.BlockSpec((2, rb, HD), lambda i: (0, i, 0)),
            pl.BlockSpec((rb, F_IN), lambda i: (i, 0)),
            full((D, HID)), full((D, HID)), full((1, HID)),
            full((HID, HID)), full((1, HID)),
            full((D, HID)), full((D, HID)),
        ],
        out_specs=[
            pl.BlockSpec((rb, D), lambda i: (i, 0)),
            pl.BlockSpec((rb, HID), lambda i: (i, 0)),
            pl.BlockSpec((rb, HID), lambda i: (i, 0)),
            pl.BlockSpec((2, rb, HD), lambda i: (0, i, 0)),
        ],
        out_shape=[
            jax.ShapeDtypeStruct((n, D), jnp.float32),
            jax.ShapeDtypeStruct((n, HID), jnp.float32),
            jax.ShapeDtypeStruct((n, HID), jnp.float32),
            jax.ShapeDtypeStruct((2, n, HD), jnp.float32),
        ],
    )


def _make_sc_gather(ch):
    mesh = plsc.VectorSubcoreMesh(core_axis_name="c", subcore_axis_name="s",
                                  num_cores=NC, num_subcores=NS)
    assert ch % GL == 0 and ch > GL

    @functools.partial(
        pl.kernel,
        out_type=jax.ShapeDtypeStruct((NW, ch, CB, HID), jnp.float32),
        mesh=mesh,
        scratch_types=[
            pltpu.VMEM((ch, CB), jnp.int32),
            pltpu.VMEM((ch, CB), jnp.int32),
            [pltpu.VMEM((CB, HID), jnp.float32) for _ in range(GL)],
            [pltpu.VMEM((CB, HID), jnp.float32) for _ in range(GL)],
            [pltpu.SemaphoreType.DMA for _ in range(GL)],
            [pltpu.SemaphoreType.DMA for _ in range(GL)],
            [pltpu.SemaphoreType.DMA for _ in range(GL)],
        ],
        compiler_params=_SC_PARAMS,
    )
    def gather(idxs_hbm, idxe_hbm, a_hbm, b_hbm, o_hbm,
               vidx_s, vidx_e, pas, pbs, gas, gbs, ws):
        cid = lax.axis_index("c")
        sid = lax.axis_index("s")
        wid = sid * NC + cid
        pltpu.sync_copy(idxs_hbm.at[wid], vidx_s)
        pltpu.sync_copy(idxe_hbm.at[wid], vidx_e)

        def start_gather(c, b):
            pltpu.async_copy(a_hbm.at[vidx_s.at[c]], pas[b], gas[b])
            pltpu.async_copy(b_hbm.at[vidx_e.at[c]], pbs[b], gbs[b])

        def wait_gather(c, b):
            pltpu.make_async_copy(a_hbm.at[vidx_s.at[c]], pas[b], gas[b]).wait()
            pltpu.make_async_copy(b_hbm.at[vidx_e.at[c]], pbs[b], gbs[b]).wait()

        def wait_write(c, b):
            pltpu.make_async_copy(pas[b], o_hbm.at[wid, c], ws[b]).wait()

        for c in range(GL - 2):
            start_gather(c, c)

        def add_rows(b):
            pa, pb = pas[b], pbs[b]

            def row(r, carry):
                for col in range(HID // 16):
                    sl = pl.ds(col * 16, 16)
                    pa[r, sl] = pa[r, sl] + pb[r, sl]
                return carry

            lax.fori_loop(0, CB, row, 0)

        def block(i, carry):
            for k in range(GL):
                c = i * GL + k
                wait_gather(c, k)
                add_rows(k)
                pltpu.async_copy(pas[k], o_hbm.at[wid, c], ws[k])
                kp = (k + GL - 2) % GL
                cp = c + GL - 2

                @pl.when(cp < ch)
                def _():
                    @pl.when(c >= 2)
                    def _():
                        wait_write(c - 2, kp)
                    start_gather(cp, kp)

            return carry

        lax.fori_loop(0, ch // GL, block, 0)
        for t in range(GL):
            c = ch - GL + t
            wait_write(c, c % GL)

    return gather


def _make_sc_scatter(ch2, n):
    mesh = plsc.VectorSubcoreMesh(core_axis_name="c", subcore_axis_name="s",
                                  num_cores=NC, num_subcores=NS)
    npad = -(-n // (NS * CB)) * NS * CB
    rows_per_tile = npad // NS
    zchunk = CB
    nslab = ch2 // SG

    @functools.partial(
        pl.kernel,
        out_type=jax.ShapeDtypeStruct((NC, npad, HD), jnp.float32),
        mesh=mesh,
        scratch_types=[
            pltpu.VMEM((SG, CB), jnp.int32),
            pltpu.VMEM((SG, CB), jnp.int32),
            pltpu.VMEM((SG, CB), jnp.float32),
            pltpu.VMEM((CB, HD), jnp.float32),
            pltpu.VMEM((CB, HD), jnp.float32),
            pltpu.VMEM_SHARED((npad, HD), jnp.float32),
            pltpu.SemaphoreType.DMA,
            pltpu.SemaphoreType.DMA,
            pltpu.SemaphoreType.DMA,
            pltpu.SemaphoreType.DMA,
        ],
        compiler_params=_SC_PARAMS,
    )
    def scatter(idxs_hbm, idxe_hbm, w_hbm, tbl_hbm, out_hbm,
                vidx_s, vidx_e, vw, rows_a, rows_b, accum,
                gsem_a, gsem_b, ssem_a, ssem_b):
        cid = lax.axis_index("c")
        sid = lax.axis_index("s")
        tbl = tbl_hbm.at[cid]

        zv = jnp.zeros((16,), jnp.float32)

        def zrow(r, carry):
            for c in range(HD // 16):
                rows_a[r, pl.ds(c * 16, 16)] = zv
            return carry

        lax.fori_loop(0, zchunk, zrow, 0)
        base = sid * rows_per_tile
        for q in range(rows_per_tile // zchunk):
            pltpu.sync_copy(rows_a.at[pl.ds(0, zchunk)],
                            accum.at[pl.ds(base + q * zchunk, zchunk)])
        plsc.subcore_barrier()

        def scale(rows, j):
            def per_group(g, carry):
                wv = vw[j, pl.ds(g * 16, 16)]
                for i in range(16):
                    e = g * 16 + i
                    w = wv[i]
                    for c in range(HD // 16):
                        sl = pl.ds(c * 16, 16)
                        rows[e, sl] = rows[e, sl] * w
                return carry
            lax.fori_loop(0, CB // 16, per_group, 0)

        def load_slab(s):
            pltpu.sync_copy(idxs_hbm.at[sid, pl.ds(s * SG, SG)], vidx_s)
            pltpu.sync_copy(idxe_hbm.at[sid, pl.ds(s * SG, SG)], vidx_e)
            pltpu.sync_copy(w_hbm.at[sid, pl.ds(s * SG, SG)], vw)

        def start_gathers(j):
            pltpu.async_copy(tbl.at[vidx_e.at[j]], rows_a, gsem_a)
            pltpu.async_copy(tbl.at[vidx_s.at[j]], rows_b, gsem_b)

        def wait_scatters(j):
            pltpu.make_async_copy(
                rows_a, accum.at[vidx_s.at[j]], ssem_a).wait()
            pltpu.make_async_copy(
                rows_b, accum.at[vidx_e.at[j]], ssem_b).wait()

        load_slab(0)
        start_gathers(0)

        def chunk(j, carry):
            pltpu.make_async_copy(tbl.at[vidx_e.at[j]], rows_a, gsem_a).wait()
            scale(rows_a, j)
            pltpu.async_copy(rows_a, accum.at[vidx_s.at[j]], ssem_a, add=True)
            pltpu.make_async_copy(tbl.at[vidx_s.at[j]], rows_b, gsem_b).wait()
            scale(rows_b, j)
            pltpu.async_copy(rows_b, accum.at[vidx_e.at[j]], ssem_b, add=True)

            @pl.when(j + 1 < SG)
            def _():
                wait_scatters(j)
                start_gathers(j + 1)

            return carry

        def slab(s, carry):
            lax.fori_loop(0, SG, chunk, 0)
            wait_scatters(SG - 1)

            @pl.when(s + 1 < nslab)
            def _():
                load_slab(s + 1)
                start_gathers(0)

            return carry

        lax.fori_loop(0, nslab, slab, 0)
        plsc.subcore_barrier()

        for q in range(rows_per_tile // zchunk):
            r0 = base + q * zchunk
            pltpu.sync_copy(accum.at[pl.ds(r0, zchunk)],
                            rows_a.at[pl.ds(0, zchunk)])
            pltpu.sync_copy(rows_a.at[pl.ds(0, zchunk)],
                            out_hbm.at[cid, pl.ds(r0, zchunk)])

    return scatter


def kernel(x, edge_index, Wenc, benc, We1, be1, We2, be2, We3, be3,
           Wn1, bn1, Wn2, bn2):
    n = x.shape[0]
    e = edge_index.shape[1]
    import math
    cmul = math.lcm(NW * GL, NS * SG)
    chunks_total = -(-e // (CB * cmul)) * cmul
    epad = chunks_total * CB
    ch = chunks_total // NW
    ch2 = chunks_total // NS

    pad = epad - e
    start = jnp.concatenate([edge_index[0], jnp.zeros((pad,), jnp.int32)])
    end = jnp.concatenate([edge_index[1], jnp.zeros((pad,), jnp.int32)])
    idx_s = start.reshape(NW, ch, CB)
    idx_e = end.reshape(NW, ch, CB)
    idx_s2 = start.reshape(NS, ch2, CB)
    idx_e2 = end.reshape(NS, ch2, CB)
    mask = (jnp.arange(epad, dtype=jnp.int32) < e).astype(jnp.float32)
    mask = mask.reshape(epad, 1)

    benc_r = benc.reshape(1, HID)
    be1_r = be1.reshape(1, HID)
    be2_r = be2.reshape(1, HID)
    be3_r = be3.reshape(1, 1)
    bn1_r = bn1.reshape(1, HID)
    bn2_r = bn2.reshape(1, HID)
    we3_r = We3.reshape(1, HID)
    Wa, Wb = We1[:D], We1[D:]
    Wn1_s, Wn1_a = Wn1[:D], Wn1[D:]

    npad = -(-n // (NS * CB)) * NS * CB
    encode = _make_encode(n, 2000)
    edge_w = _make_edge(epad, 4096, sigmoid=True)
    edge_logit = _make_edge(epad, 4096, sigmoid=False)
    node = _make_node(n, npad, 2000)
    sc_gather = _make_sc_gather(ch)
    sc_scatter = _make_sc_scatter(ch2, n)

    h, A, B, tbl = encode(x, Wenc, benc_r, Wa, Wb)

    for _ in range(N_ITERS):
        s = sc_gather(idx_s, idx_e, A, B)
        w = edge_w(s.reshape(epad, HID),
                   be1_r, We2, be2_r, we3_r, be3_r, mask)
        parts = sc_scatter(idx_s2, idx_e2, w.reshape(NS, ch2, CB), tbl)
        h, A, B, tbl = node(h, parts, x, Wn1_s, Wn1_a, bn1_r, Wn2, bn2_r,
                            Wa, Wb)

    s = sc_gather(idx_s, idx_e, A, B)
    logits = edge_logit(s.reshape(epad, HID),
                        be1_r, We2, be2_r, we3_r, be3_r, mask)
    return logits.reshape(epad)[:e]


# speedup vs baseline: 2.4909x; 1.1057x over previous
import functools

import jax
import jax.numpy as jnp
from jax import lax
from jax.experimental import pallas as pl
from jax.experimental.pallas import tpu as pltpu
from jax.experimental.pallas import tpu_sc as plsc

F_IN = 128
HID = 64
D = F_IN + HID
HD = D // 2
N_ITERS = 4

NC = 2
NS = 16
NW = NC * NS
CB = 128
SG = 16
GL = 5

_SC_PARAMS = pltpu.CompilerParams(use_tc_tiling_on_sc=False,
                                  needs_layout_passes=False)


def _split_tables(h2):
    return jnp.stack([h2[:, :HD], h2[:, HD:]], axis=0)


def _encode_body(x_ref, wenc_ref, benc_ref, wa_ref, wb_ref,
                 h_ref, a_ref, b_ref, tbl_ref):
    x = x_ref[...]
    henc = jnp.maximum(
        jnp.dot(x, wenc_ref[...], preferred_element_type=jnp.float32)
        + benc_ref[...], 0.0)
    h = jnp.concatenate([henc, x], axis=1)
    h_ref[...] = h
    a_ref[...] = jnp.dot(h, wa_ref[...], preferred_element_type=jnp.float32)
    b_ref[...] = jnp.dot(h, wb_ref[...], preferred_element_type=jnp.float32)
    tbl_ref[...] = _split_tables(h)


def _make_encode(n, rb):
    grid = n // rb
    full = lambda shape: pl.BlockSpec(shape, lambda i: (0,) * len(shape))
    return pl.pallas_call(
        _encode_body,
        grid=(grid,),
        in_specs=[
            pl.BlockSpec((rb, F_IN), lambda i: (i, 0)),
            full((F_IN, HID)), full((1, HID)),
            full((D, HID)), full((D, HID)),
        ],
        out_specs=[
            pl.BlockSpec((rb, D), lambda i: (i, 0)),
            pl.BlockSpec((rb, HID), lambda i: (i, 0)),
            pl.BlockSpec((rb, HID), lambda i: (i, 0)),
            pl.BlockSpec((2, rb, HD), lambda i: (0, i, 0)),
        ],
        out_shape=[
            jax.ShapeDtypeStruct((n, D), jnp.float32),
            jax.ShapeDtypeStruct((n, HID), jnp.float32),
            jax.ShapeDtypeStruct((n, HID), jnp.float32),
            jax.ShapeDtypeStruct((2, n, HD), jnp.float32),
        ],
    )


def _edge_body(sigmoid, s_ref, be1_ref, w2_ref, be2_ref, w3_ref,
               be3_ref, mask_ref, out_ref):
    z1 = jnp.maximum(s_ref[...] + be1_ref[...], 0.0)
    z2 = jnp.maximum(
        jnp.dot(z1, w2_ref[...], preferred_element_type=jnp.float32)
        + be2_ref[...], 0.0)
    logit = jnp.sum(z2 * w3_ref[...], axis=1, keepdims=True) + be3_ref[...]
    if sigmoid:
        out_ref[...] = jax.nn.sigmoid(logit) * mask_ref[...]
    else:
        out_ref[...] = logit


def _make_edge(epad, rb, sigmoid):
    grid = epad // rb
    full = lambda shape: pl.BlockSpec(shape, lambda i: (0,) * len(shape))
    return pl.pallas_call(
        functools.partial(_edge_body, sigmoid),
        grid=(grid,),
        in_specs=[
            pl.BlockSpec((rb, HID), lambda i: (i, 0)),
            full((1, HID)), full((HID, HID)), full((1, HID)),
            full((1, HID)), full((1, 1)),
            pl.BlockSpec((rb, 1), lambda i: (i, 0)),
        ],
        out_specs=pl.BlockSpec((rb, 1), lambda i: (i, 0)),
        out_shape=jax.ShapeDtypeStruct((epad, 1), jnp.float32),
    )


def _node_body(h_ref, p_ref, x_ref, wn1s_ref, wn1a_ref, bn1_ref, wn2_ref,
               bn2_ref, wa_ref, wb_ref, h2_ref, a_ref, b_ref, tbl_ref):
    h = h_ref[...]
    aggr = jnp.concatenate([p_ref[0], p_ref[1]], axis=1)
    n1 = jnp.maximum(
        jnp.dot(h, wn1s_ref[...], preferred_element_type=jnp.float32)
        + jnp.dot(aggr, wn1a_ref[...], preferred_element_type=jnp.float32)
        + bn1_ref[...], 0.0)
    hn = jnp.dot(n1, wn2_ref[...], preferred_element_type=jnp.float32) + bn2_ref[...]
    h2 = jnp.concatenate([hn, x_ref[...]], axis=1) + h
    h2_ref[...] = h2
    a_ref[...] = jnp.dot(h2, wa_ref[...], preferred_element_type=jnp.float32)
    b_ref[...] = jnp.dot(h2, wb_ref[...], preferred_element_type=jnp.float32)
    tbl_ref[...] = _split_tables(h2)


def _make_node(n, npad, rb):
    grid = n // rb
    full = lambda shape: pl.BlockSpec(shape, lambda i: (0,) * len(shape))
    return pl.pallas_call(
        _node_body,
        grid=(grid,),
        in_specs=[
            pl.BlockSpec((rb, D), lambda i: (i, 0)),
            pl.BlockSpec((2, rb, HD), lambda i: (0, i, 0)),
            pl.BlockSpec((rb, F_IN), lambda i: (i, 0)),
            full((D, HID)), full((D, HID)), full((1, HID)),
            full((HID, HID)), full((1, HID)),
            full((D, HID)), full((D, HID)),
        ],
        out_specs=[
            pl.BlockSpec((rb, D), lambda i: (i, 0)),
            pl.BlockSpec((rb, HID), lambda i: (i, 0)),
            pl.BlockSpec((rb, HID), lambda i: (i, 0)),
            pl.BlockSpec((2, rb, HD), lambda i: (0, i, 0)),
        ],
        out_shape=[
            jax.ShapeDtypeStruct((n, D), jnp.float32),
            jax.ShapeDtypeStruct((n, HID), jnp.float32),
            jax.ShapeDtypeStruct((n, HID), jnp.float32),
            jax.ShapeDtypeStruct((2, n, HD), jnp.float32),
        ],
    )


def _make_sc_gather(ch):
    mesh = plsc.VectorSubcoreMesh(core_axis_name="c", subcore_axis_name="s",
                                  num_cores=NC, num_subcores=NS)
    assert ch % GL == 0 and ch > GL

    @functools.partial(
        pl.kernel,
        out_type=jax.ShapeDtypeStruct((NW, ch, CB, HID), jnp.float32),
        mesh=mesh,
        scratch_types=[
            pltpu.VMEM((ch, CB), jnp.int32),
            pltpu.VMEM((ch, CB), jnp.int32),
            [pltpu.VMEM((CB, HID), jnp.float32) for _ in range(GL)],
            [pltpu.VMEM((CB, HID), jnp.float32) for _ in range(GL)],
            [pltpu.SemaphoreType.DMA for _ in range(GL)],
            [pltpu.SemaphoreType.DMA for _ in range(GL)],
            [pltpu.SemaphoreType.DMA for _ in range(GL)],
        ],
        compiler_params=_SC_PARAMS,
    )
    def gather(idxs_hbm, idxe_hbm, a_hbm, b_hbm, o_hbm,
               vidx_s, vidx_e, pas, pbs, gas, gbs, ws):
        cid = lax.axis_index("c")
        sid = lax.axis_index("s")
        wid = sid * NC + cid
        pltpu.sync_copy(idxs_hbm.at[wid], vidx_s)
        pltpu.sync_copy(idxe_hbm.at[wid], vidx_e)

        def start_gather(c, b):
            pltpu.async_copy(a_hbm.at[vidx_s.at[c]], pas[b], gas[b])
            pltpu.async_copy(b_hbm.at[vidx_e.at[c]], pbs[b], gbs[b])

        def wait_gather(c, b):
            pltpu.make_async_copy(a_hbm.at[vidx_s.at[c]], pas[b], gas[b]).wait()
            pltpu.make_async_copy(b_hbm.at[vidx_e.at[c]], pbs[b], gbs[b]).wait()

        def wait_write(c, b):
            pltpu.make_async_copy(pas[b], o_hbm.at[wid, c], ws[b]).wait()

        for c in range(GL - 2):
            start_gather(c, c)

        def add_rows(b):
            pa, pb = pas[b], pbs[b]

            @plsc.parallel_loop(0, CB, unroll=4)
            def _(r):
                for col in range(HID // 16):
                    sl = pl.ds(col * 16, 16)
                    pa[r, sl] = pa[r, sl] + pb[r, sl]

        def block(i, carry):
            for k in range(GL):
                c = i * GL + k
                wait_gather(c, k)
                add_rows(k)
                pltpu.async_copy(pas[k], o_hbm.at[wid, c], ws[k])
                kp = (k + GL - 2) % GL
                cp = c + GL - 2

                @pl.when(cp < ch)
                def _():
                    @pl.when(c >= 2)
                    def _():
                        wait_write(c - 2, kp)
                    start_gather(cp, kp)

            return carry

        lax.fori_loop(0, ch // GL, block, 0)
        for t in range(GL):
            c = ch - GL + t
            wait_write(c, c % GL)

    return gather


def _make_sc_scatter(ch2, n):
    mesh = plsc.VectorSubcoreMesh(core_axis_name="c", subcore_axis_name="s",
                                  num_cores=NC, num_subcores=NS)
    npad = -(-n // (NS * CB)) * NS * CB
    rows_per_tile = npad // NS
    zchunk = CB
    nslab = ch2 // SG

    @functools.partial(
        pl.kernel,
        out_type=jax.ShapeDtypeStruct((NC, npad, HD), jnp.float32),
        mesh=mesh,
        scratch_types=[
            pltpu.VMEM((SG, CB), jnp.int32),
            pltpu.VMEM((SG, CB), jnp.int32),
            pltpu.VMEM((SG, CB), jnp.float32),
            [pltpu.VMEM((CB, HD), jnp.float32) for _ in range(4)],
            pltpu.VMEM_SHARED((npad, HD), jnp.float32),
            [pltpu.SemaphoreType.DMA for _ in range(4)],
            [pltpu.SemaphoreType.DMA for _ in range(4)],
        ],
        compiler_params=_SC_PARAMS,
    )
    def scatter(idxs_hbm, idxe_hbm, w_hbm, tbl_hbm, out_hbm,
                vidx_s, vidx_e, vw, bufs, accum, gsems, ssems):
        cid = lax.axis_index("c")
        sid = lax.axis_index("s")
        tbl = tbl_hbm.at[cid]
        rows_a = bufs[0]

        zv = jnp.zeros((16,), jnp.float32)

        def zrow(r, carry):
            for c in range(HD // 16):
                rows_a[r, pl.ds(c * 16, 16)] = zv
            return carry

        lax.fori_loop(0, zchunk, zrow, 0)
        base = sid * rows_per_tile
        for q in range(rows_per_tile // zchunk):
            pltpu.sync_copy(rows_a.at[pl.ds(0, zchunk)],
                            accum.at[pl.ds(base + q * zchunk, zchunk)])
        plsc.subcore_barrier()

        def scale(rows, j):
            @plsc.parallel_loop(0, CB // 16, unroll=2)
            def _(g):
                wv = vw[j, pl.ds(g * 16, 16)]
                for i in range(16):
                    e = g * 16 + i
                    w = wv[i]
                    for c in range(HD // 16):
                        sl = pl.ds(c * 16, 16)
                        rows[e, sl] = rows[e, sl] * w

        def load_slab(s):
            pltpu.sync_copy(idxs_hbm.at[sid, pl.ds(s * SG, SG)], vidx_s)
            pltpu.sync_copy(idxe_hbm.at[sid, pl.ds(s * SG, SG)], vidx_e)
            pltpu.sync_copy(w_hbm.at[sid, pl.ds(s * SG, SG)], vw)

        def gidx(j, d):
            return vidx_e.at[j] if d == 0 else vidx_s.at[j]

        def sidx(j, d):
            return vidx_s.at[j] if d == 0 else vidx_e.at[j]

        def start_task(j, d, k):
            pltpu.async_copy(tbl.at[gidx(j, d)], bufs[k], gsems[k])

        def wait_gather(k):
            pltpu.make_async_copy(tbl.at[vidx_e.at[0]], bufs[k],
                                  gsems[k]).wait()

        def wait_scatter(k):
            pltpu.make_async_copy(bufs[k], accum.at[vidx_e.at[0]],
                                  ssems[k]).wait()

        def slab(s, carry):
            start_task(0, 0, 0)
            start_task(0, 1, 1)

            def body(i, carry2):
                j0 = 2 * i
                for k in range(4):
                    j = j0 if k < 2 else j0 + 1
                    d = k % 2
                    wait_gather(k)
                    scale(bufs[k], j)
                    pltpu.async_copy(bufs[k], accum.at[sidx(j, d)],
                                     ssems[k], add=True)
                    kp = (k + 2) % 4
                    jp = j0 + 1 if k < 2 else j0 + 2
                    t = 4 * i + k

                    @pl.when(jp < SG)
                    def _():
                        @pl.when(t >= 2)
                        def _():
                            wait_scatter(kp)
                        start_task(jp, d, kp)

                return carry2

            lax.fori_loop(0, SG // 2, body, 0)
            for k in range(4):
                wait_scatter(k)

            @pl.when(s + 1 < nslab)
            def _():
                load_slab(s + 1)

            return carry

        load_slab(0)
        lax.fori_loop(0, nslab, slab, 0)
        plsc.subcore_barrier()

        for q in range(rows_per_tile // zchunk):
            r0 = base + q * zchunk
            pltpu.sync_copy(accum.at[pl.ds(r0, zchunk)],
                            rows_a.at[pl.ds(0, zchunk)])
            pltpu.sync_copy(rows_a.at[pl.ds(0, zchunk)],
                            out_hbm.at[cid, pl.ds(r0, zchunk)])

    return scatter


def kernel(x, edge_index, Wenc, benc, We1, be1, We2, be2, We3, be3,
           Wn1, bn1, Wn2, bn2):
    n = x.shape[0]
    e = edge_index.shape[1]
    import math
    cmul = math.lcm(NW * GL, NS * SG)
    chunks_total = -(-e // (CB * cmul)) * cmul
    epad = chunks_total * CB
    ch = chunks_total // NW
    ch2 = chunks_total // NS

    pad = epad - e
    start = jnp.concatenate([edge_index[0], jnp.zeros((pad,), jnp.int32)])
    end = jnp.concatenate([edge_index[1], jnp.zeros((pad,), jnp.int32)])
    idx_s = start.reshape(NW, ch, CB)
    idx_e = end.reshape(NW, ch, CB)
    idx_s2 = start.reshape(NS, ch2, CB)
    idx_e2 = end.reshape(NS, ch2, CB)
    mask = (jnp.arange(epad, dtype=jnp.int32) < e).astype(jnp.float32)
    mask = mask.reshape(epad, 1)

    benc_r = benc.reshape(1, HID)
    be1_r = be1.reshape(1, HID)
    be2_r = be2.reshape(1, HID)
    be3_r = be3.reshape(1, 1)
    bn1_r = bn1.reshape(1, HID)
    bn2_r = bn2.reshape(1, HID)
    we3_r = We3.reshape(1, HID)
    Wa, Wb = We1[:D], We1[D:]
    Wn1_s, Wn1_a = Wn1[:D], Wn1[D:]

    npad = -(-n // (NS * CB)) * NS * CB
    encode = _make_encode(n, 2000)
    edge_w = _make_edge(epad, 4096, sigmoid=True)
    edge_logit = _make_edge(epad, 4096, sigmoid=False)
    node = _make_node(n, npad, 2000)
    sc_gather = _make_sc_gather(ch)
    sc_scatter = _make_sc_scatter(ch2, n)

    h, A, B, tbl = encode(x, Wenc, benc_r, Wa, Wb)

    for _ in range(N_ITERS):
        s = sc_gather(idx_s, idx_e, A, B)
        w = edge_w(s.reshape(epad, HID),
                   be1_r, We2, be2_r, we3_r, be3_r, mask)
        parts = sc_scatter(idx_s2, idx_e2, w.reshape(NS, ch2, CB), tbl)
        h, A, B, tbl = node(h, parts, x, Wn1_s, Wn1_a, bn1_r, Wn2, bn2_r,
                            Wa, Wb)

    s = sc_gather(idx_s, idx_e, A, B)
    logits = edge_logit(s.reshape(epad, HID),
                        be1_r, We2, be2_r, we3_r, be3_r, mask)
    return logits.reshape(epad)[:e]
